# baseline (speedup 1.0000x reference)
import jax
import jax.numpy as jnp
from jax.experimental import pallas as pl
from jax.experimental.pallas import tpu as pltpu
from jax.experimental.pallas import tpu_sc as plsc

H = 12
HD = 64
BLK = 32
NBLK = 4
QBS = 16
SEQ_TILE = 256

_NC = 2
_NS = 16
_NW = _NC * _NS


def _mm(a, b, dims=None):
    if dims is None:
        dims = (((a.ndim - 1,), (0,)), ((), ()))
    return jax.lax.dot_general(a.astype(jnp.bfloat16), b.astype(jnp.bfloat16),
                               dims, preferred_element_type=jnp.float32)


def _qkv_body(hs_ref, wq_ref, wk_ref, wv_ref, bq_ref, bk_ref, bv_ref,
              q_ref, k_ref, v_ref, qb_ref, kb_ref):
    hs = hs_ref[...]
    scale = HD ** -0.5
    q = (_mm(hs, wq_ref[...]) + bq_ref[...]) * scale
    k = _mm(hs, wk_ref[...]) + bk_ref[...]
    v = _mm(hs, wv_ref[...]) + bv_ref[...]
    q_ref[...] = q
    k_ref[...] = k
    v_ref[...] = v
    ts, dm = hs.shape
    nb = ts // BLK
    qb_ref[...] = jnp.mean(q.reshape(nb, BLK, dm), axis=1)
    kb_ref[...] = jnp.mean(k.reshape(nb, BLK, dm), axis=1)


def _scores_body(qb_ref, kb_ref, s_ref):
    nb = qb_ref.shape[0]
    for h in range(H):
        qh = qb_ref[:, h * HD:(h + 1) * HD]
        kh = kb_ref[:, h * HD:(h + 1) * HD]
        s_ref[h * nb:(h + 1) * nb, :] = _mm(qh, kh, (((1,), (1,)), ((), ())))


def _topk_sc_body(scores_hbm, top_hbm, sc_v, top_v, rotf, roti):
    rows_w = sc_v.shape[0]
    nb = sc_v.shape[1]
    nchunk = nb // 16
    wid = jax.lax.axis_index("s") * _NC + jax.lax.axis_index("c")
    base = wid * rows_w
    pltpu.sync_copy(scores_hbm.at[pl.ds(base, rows_w)], sc_v)
    lane = jax.lax.iota(jnp.int32, 16)

    def _allmax(v):
        for sh in (8, 4, 2, 1):
            rotf[0:16] = v
            rotf[16:32] = v
            v = jnp.maximum(v, rotf[sh:sh + 16])
        return v

    def _allmin_i(v):
        for sh in (8, 4, 2, 1):
            roti[0:16] = v
            roti[16:32] = v
            v = jnp.minimum(v, roti[sh:sh + 16])
        return v

    def row_fn(r, carry):
        vals = [sc_v[r, 16 * c:16 * c + 16] for c in range(nchunk)]
        gidx = [lane + 16 * c for c in range(nchunk)]
        out = jnp.zeros((16,), jnp.int32)
        for m in range(NBLK):
            mx = vals[0]
            for c in range(1, nchunk):
                mx = jnp.maximum(mx, vals[c])
            mxv = _allmax(mx)
            cmin = jnp.where(vals[0] == mxv, gidx[0], nb)
            for c in range(1, nchunk):
                cmin = jnp.minimum(cmin, jnp.where(vals[c] == mxv, gidx[c], nb))
            civ = _allmin_i(cmin)
            out = jnp.where(lane == m, civ, out)
            for c in range(nchunk):
                vals[c] = jnp.where(gidx[c] == civ, -1e30, vals[c])
        top_v[r, :] = out
        return carry

    jax.lax.fori_loop(0, rows_w, row_fn, 0)
    pltpu.sync_copy(top_v, top_hbm.at[pl.ds(base, rows_w)])


def _attn_body(top_ref, q_ref, k_ref, v_ref, o_ref):
    hp = pl.program_id(0)
    g = pl.program_id(1)
    nqb = 64
    for j in range(QBS):
        qb = g * QBS + j
        q_all = q_ref[j * BLK:(j + 1) * BLK, :]
        outs = []
        for hh in range(2):
            row = (hp * 2 + hh) * nqb + qb
            ks, vs = [], []
            for m in range(NBLK):
                idx = top_ref[row, m]
                ks.append(k_ref[pl.ds(idx * BLK, BLK), :][:, hh * HD:(hh + 1) * HD])
                vs.append(v_ref[pl.ds(idx * BLK, BLK), :][:, hh * HD:(hh + 1) * HD])
            ksel = jnp.concatenate(ks, axis=0)
            vsel = jnp.concatenate(vs, axis=0)
            qj = q_all[:, hh * HD:(hh + 1) * HD]
            s = _mm(qj, ksel, (((1,), (1,)), ((), ())))
            mx = jnp.max(s, axis=1, keepdims=True)
            e = jnp.exp(s - mx)
            p = e / jnp.sum(e, axis=1, keepdims=True)
            outs.append(_mm(p, vsel, (((1,), (0,)), ((), ()))))
        o_ref[j * BLK:(j + 1) * BLK, :] = jnp.concatenate(outs, axis=1)


def _head_body(ctx_ref, wo_ref, bo_ref, wp_ref, bp_ref, ws_ref, wc_ref, bc_ref,
               out_ref):
    ctx = ctx_ref[...]
    ao = _mm(ctx, wo_ref[...]) + bo_ref[...]
    hp = jnp.tanh(_mm(ao, wp_ref[...]) + bp_ref[...])
    sc = jax.lax.dot_general(hp, ws_ref[...], (((1,), (1,)), ((), ())),
                             preferred_element_type=jnp.float32)
    mx = jnp.max(sc, axis=0, keepdims=True)
    e = jnp.exp(sc - mx)
    p = e / jnp.sum(e, axis=0, keepdims=True)
    pooled = jax.lax.dot_general(p, ao, (((0,), (0,)), ((), ())),
                                 preferred_element_type=jnp.float32)
    out_ref[...] = jax.lax.dot_general(pooled, wc_ref[...],
                                       (((1,), (0,)), ((), ())),
                                       preferred_element_type=jnp.float32) + bc_ref[...]


def kernel(hidden_states, attention_mask, Wq, bq, Wk, bk, Wv, bv, Wo, bo,
           Wp, bp, ws, Wc, bc):
    del attention_mask
    bsz, T, DM = hidden_states.shape
    hs = hidden_states.reshape(T, DM)
    nqb = T // BLK
    nlab = Wc.shape[1]
    bq2 = bq.reshape(1, DM)
    bk2 = bk.reshape(1, DM)
    bv2 = bv.reshape(1, DM)
    bo2 = bo.reshape(1, DM)
    bp2 = bp.reshape(1, DM)
    ws2 = ws.reshape(1, DM)
    bc2 = bc.reshape(1, nlab)

    ntile = T // SEQ_TILE
    nb_tile = SEQ_TILE // BLK
    q, k, v, qbm, kbm = pl.pallas_call(
        _qkv_body,
        grid=(ntile,),
        in_specs=[
            pl.BlockSpec((SEQ_TILE, DM), lambda t: (t, 0)),
            pl.BlockSpec((DM, DM), lambda t: (0, 0)),
            pl.BlockSpec((DM, DM), lambda t: (0, 0)),
            pl.BlockSpec((DM, DM), lambda t: (0, 0)),
            pl.BlockSpec((1, DM), lambda t: (0, 0)),
            pl.BlockSpec((1, DM), lambda t: (0, 0)),
            pl.BlockSpec((1, DM), lambda t: (0, 0)),
        ],
        out_specs=[
            pl.BlockSpec((SEQ_TILE, DM), lambda t: (t, 0)),
            pl.BlockSpec((SEQ_TILE, DM), lambda t: (t, 0)),
            pl.BlockSpec((SEQ_TILE, DM), lambda t: (t, 0)),
            pl.BlockSpec((nb_tile, DM), lambda t: (t, 0)),
            pl.BlockSpec((nb_tile, DM), lambda t: (t, 0)),
        ],
        out_shape=[
            jax.ShapeDtypeStruct((T, DM), jnp.float32),
            jax.ShapeDtypeStruct((T, DM), jnp.float32),
            jax.ShapeDtypeStruct((T, DM), jnp.float32),
            jax.ShapeDtypeStruct((nqb, DM), jnp.float32),
            jax.ShapeDtypeStruct((nqb, DM), jnp.float32),
        ],
    )(hs, Wq, Wk, Wv, bq2, bk2, bv2)

    scores = pl.pallas_call(
        _scores_body,
        out_shape=jax.ShapeDtypeStruct((H * nqb, nqb), jnp.float32),
    )(qbm, kbm)

    rows_w = (H * nqb) // _NW
    top4 = pl.kernel(
        _topk_sc_body,
        out_type=jax.ShapeDtypeStruct((H * nqb, 16), jnp.int32),
        mesh=plsc.VectorSubcoreMesh(core_axis_name="c", subcore_axis_name="s",
                                    num_cores=_NC, num_subcores=_NS),
        scratch_types=[
            pltpu.VMEM((rows_w, nqb), jnp.float32),
            pltpu.VMEM((rows_w, 16), jnp.int32),
            pltpu.VMEM((32,), jnp.float32),
            pltpu.VMEM((32,), jnp.int32),
        ],
    )(scores)

    grid_spec = pltpu.PrefetchScalarGridSpec(
        num_scalar_prefetch=1,
        grid=(H // 2, nqb // QBS),
        in_specs=[
            pl.BlockSpec((QBS * BLK, 2 * HD), lambda hp, g, top: (g, hp)),
            pl.BlockSpec((T, 2 * HD), lambda hp, g, top: (0, hp)),
            pl.BlockSpec((T, 2 * HD), lambda hp, g, top: (0, hp)),
        ],
        out_specs=pl.BlockSpec((QBS * BLK, 2 * HD), lambda hp, g, top: (g, hp)),
    )
    ctx = pl.pallas_call(
        _attn_body,
        grid_spec=grid_spec,
        out_shape=jax.ShapeDtypeStruct((T, DM), jnp.float32),
    )(top4, q, k, v)

    logits = pl.pallas_call(
        _head_body,
        out_shape=jax.ShapeDtypeStruct((1, nlab), jnp.float32),
    )(ctx, Wo, bo2, Wp, bp2, ws2, Wc, bc2)
    return logits


# speedup vs baseline: 2.6407x; 2.6407x over previous
import jax
import jax.numpy as jnp
from jax.experimental import pallas as pl
from jax.experimental.pallas import tpu as pltpu
from jax.experimental.pallas import tpu_sc as plsc

H = 12
HD = 64
BLK = 32
NBLK = 4
QBS = 16
SEQ_TILE = 256

_NC = 2
_NS = 16
_NW = _NC * _NS


def _mm(a, b, dims=None):
    if dims is None:
        dims = (((a.ndim - 1,), (0,)), ((), ()))
    return jax.lax.dot_general(a.astype(jnp.bfloat16), b.astype(jnp.bfloat16),
                               dims, preferred_element_type=jnp.float32)


def _qkv_body(hs_ref, wq_ref, wk_ref, wv_ref, bq_ref, bk_ref, bv_ref,
              q_ref, k_ref, v_ref, qb_ref, kb_ref):
    hs = hs_ref[...]
    scale = HD ** -0.5
    q = (_mm(hs, wq_ref[...]) + bq_ref[...]) * scale
    k = _mm(hs, wk_ref[...]) + bk_ref[...]
    v = _mm(hs, wv_ref[...]) + bv_ref[...]
    q_ref[...] = q
    k_ref[...] = k
    v_ref[...] = v
    ts, dm = hs.shape
    nb = ts // BLK
    qb_ref[...] = jnp.mean(q.reshape(nb, BLK, dm), axis=1)
    kb_ref[...] = jnp.mean(k.reshape(nb, BLK, dm), axis=1)


def _scores_body(qb_ref, kb_ref, s_ref):
    nb = qb_ref.shape[0]
    for h in range(H):
        qh = qb_ref[:, h * HD:(h + 1) * HD]
        kh = kb_ref[:, h * HD:(h + 1) * HD]
        s_ref[h * nb:(h + 1) * nb, :] = _mm(qh, kh, (((1,), (1,)), ((), ())))


def _topk_sc_body(scores_hbm, top_hbm, sc_v, top_v, rotf, roti):
    rows_w = sc_v.shape[0]
    nb = sc_v.shape[1]
    nchunk = nb // 16
    wid = jax.lax.axis_index("s") * _NC + jax.lax.axis_index("c")
    base = wid * rows_w
    pltpu.sync_copy(scores_hbm.at[pl.ds(base, rows_w)], sc_v)
    lane = jax.lax.iota(jnp.int32, 16)

    def _allmax(v):
        for sh in (8, 4, 2, 1):
            rotf[0:16] = v
            rotf[16:32] = v
            v = jnp.maximum(v, rotf[sh:sh + 16])
        return v

    def _allmin_i(v):
        for sh in (8, 4, 2, 1):
            roti[0:16] = v
            roti[16:32] = v
            v = jnp.minimum(v, roti[sh:sh + 16])
        return v

    def row_fn(r, carry):
        vals = [sc_v[r, 16 * c:16 * c + 16] for c in range(nchunk)]
        gidx = [lane + 16 * c for c in range(nchunk)]
        out = jnp.zeros((16,), jnp.int32)
        for m in range(NBLK):
            mx = vals[0]
            for c in range(1, nchunk):
                mx = jnp.maximum(mx, vals[c])
            mxv = _allmax(mx)
            cmin = jnp.where(vals[0] == mxv, gidx[0], nb)
            for c in range(1, nchunk):
                cmin = jnp.minimum(cmin, jnp.where(vals[c] == mxv, gidx[c], nb))
            civ = _allmin_i(cmin)
            out = jnp.where(lane == m, civ, out)
            for c in range(nchunk):
                vals[c] = jnp.where(gidx[c] == civ, -1e30, vals[c])
        top_v[r, :] = out
        return carry

    jax.lax.fori_loop(0, rows_w, row_fn, 0)
    pltpu.sync_copy(top_v, top_hbm.at[pl.ds(base, rows_w)])


def _attn_body(top_ref, q_ref, k_ref, v_ref, o_ref, st0_ref, st1_ref):
    hp = pl.program_id(0)
    g = pl.program_id(1)
    nqb = 64
    for hh, st_ref in ((0, st0_ref), (1, st1_ref)):
        kh = k_ref[:, hh * HD:(hh + 1) * HD]
        qh = q_ref[:, hh * HD:(hh + 1) * HD]
        st_ref[...] = _mm(kh, qh, (((1,), (1,)), ((), ())))
    for j in range(QBS):
        qb = g * QBS + j
        outs = []
        for hh, st_ref in ((0, st0_ref), (1, st1_ref)):
            row = (hp * 2 + hh) * nqb + qb
            ss, vs = [], []
            for m in range(NBLK):
                idx = top_ref[row, m]
                ss.append(st_ref[pl.ds(idx * BLK, BLK), j * BLK:(j + 1) * BLK])
                vs.append(v_ref[pl.ds(idx * BLK, BLK), :][:, hh * HD:(hh + 1) * HD])
            s = jnp.concatenate(ss, axis=0)
            vsel = jnp.concatenate(vs, axis=0)
            mx = jnp.max(s, axis=0, keepdims=True)
            e = jnp.exp(s - mx)
            p = e / jnp.sum(e, axis=0, keepdims=True)
            outs.append(_mm(p, vsel, (((0,), (0,)), ((), ()))))
        o_ref[j * BLK:(j + 1) * BLK, :] = jnp.concatenate(outs, axis=1)


def _head_body(ctx_ref, wo_ref, bo_ref, wp_ref, bp_ref, ws_ref, wc_ref, bc_ref,
               out_ref):
    ctx = ctx_ref[...]
    ao = _mm(ctx, wo_ref[...]) + bo_ref[...]
    hp = jnp.tanh(_mm(ao, wp_ref[...]) + bp_ref[...])
    sc = jax.lax.dot_general(hp, ws_ref[...], (((1,), (1,)), ((), ())),
                             preferred_element_type=jnp.float32)
    mx = jnp.max(sc, axis=0, keepdims=True)
    e = jnp.exp(sc - mx)
    p = e / jnp.sum(e, axis=0, keepdims=True)
    pooled = jax.lax.dot_general(p, ao, (((0,), (0,)), ((), ())),
                                 preferred_element_type=jnp.float32)
    out_ref[...] = jax.lax.dot_general(pooled, wc_ref[...],
                                       (((1,), (0,)), ((), ())),
                                       preferred_element_type=jnp.float32) + bc_ref[...]


def kernel(hidden_states, attention_mask, Wq, bq, Wk, bk, Wv, bv, Wo, bo,
           Wp, bp, ws, Wc, bc):
    del attention_mask
    bsz, T, DM = hidden_states.shape
    hs = hidden_states.reshape(T, DM)
    nqb = T // BLK
    nlab = Wc.shape[1]
    bq2 = bq.reshape(1, DM)
    bk2 = bk.reshape(1, DM)
    bv2 = bv.reshape(1, DM)
    bo2 = bo.reshape(1, DM)
    bp2 = bp.reshape(1, DM)
    ws2 = ws.reshape(1, DM)
    bc2 = bc.reshape(1, nlab)

    ntile = T // SEQ_TILE
    nb_tile = SEQ_TILE // BLK
    q, k, v, qbm, kbm = pl.pallas_call(
        _qkv_body,
        grid=(ntile,),
        in_specs=[
            pl.BlockSpec((SEQ_TILE, DM), lambda t: (t, 0)),
            pl.BlockSpec((DM, DM), lambda t: (0, 0)),
            pl.BlockSpec((DM, DM), lambda t: (0, 0)),
            pl.BlockSpec((DM, DM), lambda t: (0, 0)),
            pl.BlockSpec((1, DM), lambda t: (0, 0)),
            pl.BlockSpec((1, DM), lambda t: (0, 0)),
            pl.BlockSpec((1, DM), lambda t: (0, 0)),
        ],
        out_specs=[
            pl.BlockSpec((SEQ_TILE, DM), lambda t: (t, 0)),
            pl.BlockSpec((SEQ_TILE, DM), lambda t: (t, 0)),
            pl.BlockSpec((SEQ_TILE, DM), lambda t: (t, 0)),
            pl.BlockSpec((nb_tile, DM), lambda t: (t, 0)),
            pl.BlockSpec((nb_tile, DM), lambda t: (t, 0)),
        ],
        out_shape=[
            jax.ShapeDtypeStruct((T, DM), jnp.float32),
            jax.ShapeDtypeStruct((T, DM), jnp.float32),
            jax.ShapeDtypeStruct((T, DM), jnp.float32),
            jax.ShapeDtypeStruct((nqb, DM), jnp.float32),
            jax.ShapeDtypeStruct((nqb, DM), jnp.float32),
        ],
    )(hs, Wq, Wk, Wv, bq2, bk2, bv2)

    scores = pl.pallas_call(
        _scores_body,
        out_shape=jax.ShapeDtypeStruct((H * nqb, nqb), jnp.float32),
    )(qbm, kbm)

    rows_w = (H * nqb) // _NW
    top4 = pl.kernel(
        _topk_sc_body,
        out_type=jax.ShapeDtypeStruct((H * nqb, 16), jnp.int32),
        mesh=plsc.VectorSubcoreMesh(core_axis_name="c", subcore_axis_name="s",
                                    num_cores=_NC, num_subcores=_NS),
        scratch_types=[
            pltpu.VMEM((rows_w, nqb), jnp.float32),
            pltpu.VMEM((rows_w, 16), jnp.int32),
            pltpu.VMEM((32,), jnp.float32),
            pltpu.VMEM((32,), jnp.int32),
        ],
    )(scores)

    grid_spec = pltpu.PrefetchScalarGridSpec(
        num_scalar_prefetch=1,
        grid=(H // 2, nqb // QBS),
        in_specs=[
            pl.BlockSpec((QBS * BLK, 2 * HD), lambda hp, g, top: (g, hp)),
            pl.BlockSpec((T, 2 * HD), lambda hp, g, top: (0, hp)),
            pl.BlockSpec((T, 2 * HD), lambda hp, g, top: (0, hp)),
        ],
        out_specs=pl.BlockSpec((QBS * BLK, 2 * HD), lambda hp, g, top: (g, hp)),
        scratch_shapes=[
            pltpu.VMEM((T, QBS * BLK), jnp.float32),
            pltpu.VMEM((T, QBS * BLK), jnp.float32),
        ],
    )
    ctx = pl.pallas_call(
        _attn_body,
        grid_spec=grid_spec,
        out_shape=jax.ShapeDtypeStruct((T, DM), jnp.float32),
    )(top4, q, k, v)

    logits = pl.pallas_call(
        _head_body,
        out_shape=jax.ShapeDtypeStruct((1, nlab), jnp.float32),
    )(ctx, Wo, bo2, Wp, bp2, ws2, Wc, bc2)
    return logits


# speedup vs baseline: 2.7574x; 1.0442x over previous
import jax
import jax.numpy as jnp
from jax.experimental import pallas as pl
from jax.experimental.pallas import tpu as pltpu
from jax.experimental.pallas import tpu_sc as plsc

H = 12
HD = 64
BLK = 32
NBLK = 4
QBS = 32
SEQ_TILE = 256

_NC = 2
_NS = 16
_NW = _NC * _NS


def _mm(a, b, dims=None):
    if dims is None:
        dims = (((a.ndim - 1,), (0,)), ((), ()))
    return jax.lax.dot_general(a.astype(jnp.bfloat16), b.astype(jnp.bfloat16),
                               dims, preferred_element_type=jnp.float32)


def _qkv_body(hs_ref, wq_ref, wk_ref, wv_ref, bq_ref, bk_ref, bv_ref,
              q_ref, k_ref, v_ref, qb_ref, kb_ref):
    hs = hs_ref[...]
    scale = HD ** -0.5
    q = (_mm(hs, wq_ref[...]) + bq_ref[...]) * scale
    k = _mm(hs, wk_ref[...]) + bk_ref[...]
    v = _mm(hs, wv_ref[...]) + bv_ref[...]
    q_ref[...] = q
    k_ref[...] = k
    v_ref[...] = v
    ts, dm = hs.shape
    nb = ts // BLK
    qb_ref[...] = jnp.mean(q.reshape(nb, BLK, dm), axis=1)
    kb_ref[...] = jnp.mean(k.reshape(nb, BLK, dm), axis=1)


def _scores_body(qb_ref, kb_ref, s_ref):
    nb = qb_ref.shape[0]
    for h in range(H):
        qh = qb_ref[:, h * HD:(h + 1) * HD]
        kh = kb_ref[:, h * HD:(h + 1) * HD]
        s_ref[h * nb:(h + 1) * nb, :] = _mm(qh, kh, (((1,), (1,)), ((), ())))


def _topk_sc_body(scores_hbm, top_hbm, sc_v, top_v, rotf, roti):
    rows_w = sc_v.shape[0]
    nb = sc_v.shape[1]
    nchunk = nb // 16
    wid = jax.lax.axis_index("s") * _NC + jax.lax.axis_index("c")
    base = wid * rows_w
    pltpu.sync_copy(scores_hbm.at[pl.ds(base, rows_w)], sc_v)
    lane = jax.lax.iota(jnp.int32, 16)

    def _allmax(v):
        for sh in (8, 4, 2, 1):
            rotf[0:16] = v
            rotf[16:32] = v
            v = jnp.maximum(v, rotf[sh:sh + 16])
        return v

    def _allmin_i(v):
        for sh in (8, 4, 2, 1):
            roti[0:16] = v
            roti[16:32] = v
            v = jnp.minimum(v, roti[sh:sh + 16])
        return v

    def row_fn(r, carry):
        vals = [sc_v[r, 16 * c:16 * c + 16] for c in range(nchunk)]
        gidx = [lane + 16 * c for c in range(nchunk)]
        out = jnp.zeros((16,), jnp.int32)
        for m in range(NBLK):
            mx = vals[0]
            for c in range(1, nchunk):
                mx = jnp.maximum(mx, vals[c])
            mxv = _allmax(mx)
            cmin = jnp.where(vals[0] == mxv, gidx[0], nb)
            for c in range(1, nchunk):
                cmin = jnp.minimum(cmin, jnp.where(vals[c] == mxv, gidx[c], nb))
            civ = _allmin_i(cmin)
            out = jnp.where(lane == m, civ, out)
            for c in range(nchunk):
                vals[c] = jnp.where(gidx[c] == civ, -1e30, vals[c])
        top_v[r, :] = out
        return carry

    jax.lax.fori_loop(0, rows_w, row_fn, 0)
    pltpu.sync_copy(top_v, top_hbm.at[pl.ds(base, rows_w)])


def _attn_body(top_ref, q_ref, k_ref, v_ref, o_ref, st0_ref, st1_ref):
    hp = pl.program_id(0)
    g = pl.program_id(1)
    nqb = 64
    for hh, st_ref in ((0, st0_ref), (1, st1_ref)):
        kh = k_ref[:, hh * HD:(hh + 1) * HD]
        qh = q_ref[:, hh * HD:(hh + 1) * HD]
        st_ref[...] = _mm(kh, qh, (((1,), (1,)), ((), ())))
    for j in range(QBS):
        qb = g * QBS + j
        outs = []
        for hh, st_ref in ((0, st0_ref), (1, st1_ref)):
            row = (hp * 2 + hh) * nqb + qb
            ss, vs = [], []
            for m in range(NBLK):
                idx = top_ref[row, m]
                ss.append(st_ref[pl.ds(idx * BLK, BLK), j * BLK:(j + 1) * BLK])
                vs.append(v_ref[pl.ds(idx * BLK, BLK), :][:, hh * HD:(hh + 1) * HD])
            s = jnp.concatenate(ss, axis=0)
            vsel = jnp.concatenate(vs, axis=0)
            mx = jnp.max(s, axis=0, keepdims=True)
            e = jnp.exp(s - mx)
            p = e / jnp.sum(e, axis=0, keepdims=True)
            outs.append(_mm(p, vsel, (((0,), (0,)), ((), ()))))
        o_ref[j * BLK:(j + 1) * BLK, :] = jnp.concatenate(outs, axis=1)


def _head_body(ctx_ref, wo_ref, bo_ref, wp_ref, bp_ref, ws_ref, wc_ref, bc_ref,
               out_ref):
    ctx = ctx_ref[...]
    ao = _mm(ctx, wo_ref[...]) + bo_ref[...]
    hp = jnp.tanh(_mm(ao, wp_ref[...]) + bp_ref[...])
    sc = jax.lax.dot_general(hp, ws_ref[...], (((1,), (1,)), ((), ())),
                             preferred_element_type=jnp.float32)
    mx = jnp.max(sc, axis=0, keepdims=True)
    e = jnp.exp(sc - mx)
    p = e / jnp.sum(e, axis=0, keepdims=True)
    pooled = jax.lax.dot_general(p, ao, (((0,), (0,)), ((), ())),
                                 preferred_element_type=jnp.float32)
    out_ref[...] = jax.lax.dot_general(pooled, wc_ref[...],
                                       (((1,), (0,)), ((), ())),
                                       preferred_element_type=jnp.float32) + bc_ref[...]


def kernel(hidden_states, attention_mask, Wq, bq, Wk, bk, Wv, bv, Wo, bo,
           Wp, bp, ws, Wc, bc):
    del attention_mask
    bsz, T, DM = hidden_states.shape
    hs = hidden_states.reshape(T, DM)
    nqb = T // BLK
    nlab = Wc.shape[1]
    bq2 = bq.reshape(1, DM)
    bk2 = bk.reshape(1, DM)
    bv2 = bv.reshape(1, DM)
    bo2 = bo.reshape(1, DM)
    bp2 = bp.reshape(1, DM)
    ws2 = ws.reshape(1, DM)
    bc2 = bc.reshape(1, nlab)

    ntile = T // SEQ_TILE
    nb_tile = SEQ_TILE // BLK
    q, k, v, qbm, kbm = pl.pallas_call(
        _qkv_body,
        grid=(ntile,),
        in_specs=[
            pl.BlockSpec((SEQ_TILE, DM), lambda t: (t, 0)),
            pl.BlockSpec((DM, DM), lambda t: (0, 0)),
            pl.BlockSpec((DM, DM), lambda t: (0, 0)),
            pl.BlockSpec((DM, DM), lambda t: (0, 0)),
            pl.BlockSpec((1, DM), lambda t: (0, 0)),
            pl.BlockSpec((1, DM), lambda t: (0, 0)),
            pl.BlockSpec((1, DM), lambda t: (0, 0)),
        ],
        out_specs=[
            pl.BlockSpec((SEQ_TILE, DM), lambda t: (t, 0)),
            pl.BlockSpec((SEQ_TILE, DM), lambda t: (t, 0)),
            pl.BlockSpec((SEQ_TILE, DM), lambda t: (t, 0)),
            pl.BlockSpec((nb_tile, DM), lambda t: (t, 0)),
            pl.BlockSpec((nb_tile, DM), lambda t: (t, 0)),
        ],
        out_shape=[
            jax.ShapeDtypeStruct((T, DM), jnp.float32),
            jax.ShapeDtypeStruct((T, DM), jnp.float32),
            jax.ShapeDtypeStruct((T, DM), jnp.float32),
            jax.ShapeDtypeStruct((nqb, DM), jnp.float32),
            jax.ShapeDtypeStruct((nqb, DM), jnp.float32),
        ],
    )(hs, Wq, Wk, Wv, bq2, bk2, bv2)

    scores = pl.pallas_call(
        _scores_body,
        out_shape=jax.ShapeDtypeStruct((H * nqb, nqb), jnp.float32),
    )(qbm, kbm)

    rows_w = (H * nqb) // _NW
    top4 = pl.kernel(
        _topk_sc_body,
        out_type=jax.ShapeDtypeStruct((H * nqb, 16), jnp.int32),
        mesh=plsc.VectorSubcoreMesh(core_axis_name="c", subcore_axis_name="s",
                                    num_cores=_NC, num_subcores=_NS),
        scratch_types=[
            pltpu.VMEM((rows_w, nqb), jnp.float32),
            pltpu.VMEM((rows_w, 16), jnp.int32),
            pltpu.VMEM((32,), jnp.float32),
            pltpu.VMEM((32,), jnp.int32),
        ],
    )(scores)

    grid_spec = pltpu.PrefetchScalarGridSpec(
        num_scalar_prefetch=1,
        grid=(H // 2, nqb // QBS),
        in_specs=[
            pl.BlockSpec((QBS * BLK, 2 * HD), lambda hp, g, top: (g, hp)),
            pl.BlockSpec((T, 2 * HD), lambda hp, g, top: (0, hp)),
            pl.BlockSpec((T, 2 * HD), lambda hp, g, top: (0, hp)),
        ],
        out_specs=pl.BlockSpec((QBS * BLK, 2 * HD), lambda hp, g, top: (g, hp)),
        scratch_shapes=[
            pltpu.VMEM((T, QBS * BLK), jnp.float32),
            pltpu.VMEM((T, QBS * BLK), jnp.float32),
        ],
    )
    ctx = pl.pallas_call(
        _attn_body,
        grid_spec=grid_spec,
        out_shape=jax.ShapeDtypeStruct((T, DM), jnp.float32),
    )(top4, q, k, v)

    logits = pl.pallas_call(
        _head_body,
        out_shape=jax.ShapeDtypeStruct((1, nlab), jnp.float32),
    )(ctx, Wo, bo2, Wp, bp2, ws2, Wc, bc2)
    return logits


# speedup vs baseline: 2.7800x; 1.0082x over previous
import jax
import jax.numpy as jnp
from jax.experimental import pallas as pl
from jax.experimental.pallas import tpu as pltpu
from jax.experimental.pallas import tpu_sc as plsc

H = 12
HD = 64
BLK = 32
NBLK = 4
QBS = 64
SEQ_TILE = 256

_NC = 2
_NS = 16
_NW = _NC * _NS


def _mm(a, b, dims=None):
    if dims is None:
        dims = (((a.ndim - 1,), (0,)), ((), ()))
    return jax.lax.dot_general(a.astype(jnp.bfloat16), b.astype(jnp.bfloat16),
                               dims, preferred_element_type=jnp.float32)


def _qkv_body(hs_ref, wq_ref, wk_ref, wv_ref, bq_ref, bk_ref, bv_ref,
              q_ref, k_ref, v_ref, qb_ref, kb_ref):
    hs = hs_ref[...]
    scale = HD ** -0.5
    q = (_mm(hs, wq_ref[...]) + bq_ref[...]) * scale
    k = _mm(hs, wk_ref[...]) + bk_ref[...]
    v = _mm(hs, wv_ref[...]) + bv_ref[...]
    q_ref[...] = q
    k_ref[...] = k
    v_ref[...] = v
    ts, dm = hs.shape
    nb = ts // BLK
    qb_ref[...] = jnp.mean(q.reshape(nb, BLK, dm), axis=1)
    kb_ref[...] = jnp.mean(k.reshape(nb, BLK, dm), axis=1)


def _scores_body(qb_ref, kb_ref, s_ref):
    nb = qb_ref.shape[0]
    for h in range(H):
        qh = qb_ref[:, h * HD:(h + 1) * HD]
        kh = kb_ref[:, h * HD:(h + 1) * HD]
        s_ref[h * nb:(h + 1) * nb, :] = _mm(qh, kh, (((1,), (1,)), ((), ())))


def _topk_sc_body(scores_hbm, top_hbm, sc_v, top_v, rotf, roti):
    rows_w = sc_v.shape[0]
    nb = sc_v.shape[1]
    nchunk = nb // 16
    wid = jax.lax.axis_index("s") * _NC + jax.lax.axis_index("c")
    base = wid * rows_w
    pltpu.sync_copy(scores_hbm.at[pl.ds(base, rows_w)], sc_v)
    lane = jax.lax.iota(jnp.int32, 16)

    def _allmax(v):
        for sh in (8, 4, 2, 1):
            rotf[0:16] = v
            rotf[16:32] = v
            v = jnp.maximum(v, rotf[sh:sh + 16])
        return v

    def _allmin_i(v):
        for sh in (8, 4, 2, 1):
            roti[0:16] = v
            roti[16:32] = v
            v = jnp.minimum(v, roti[sh:sh + 16])
        return v

    def row_fn(r, carry):
        vals = [sc_v[r, 16 * c:16 * c + 16] for c in range(nchunk)]
        gidx = [lane + 16 * c for c in range(nchunk)]
        out = jnp.zeros((16,), jnp.int32)
        for m in range(NBLK):
            mx = vals[0]
            for c in range(1, nchunk):
                mx = jnp.maximum(mx, vals[c])
            mxv = _allmax(mx)
            cmin = jnp.where(vals[0] == mxv, gidx[0], nb)
            for c in range(1, nchunk):
                cmin = jnp.minimum(cmin, jnp.where(vals[c] == mxv, gidx[c], nb))
            civ = _allmin_i(cmin)
            out = jnp.where(lane == m, civ, out)
            for c in range(nchunk):
                vals[c] = jnp.where(gidx[c] == civ, -1e30, vals[c])
        top_v[r, :] = out
        return carry

    jax.lax.fori_loop(0, rows_w, row_fn, 0)
    pltpu.sync_copy(top_v, top_hbm.at[pl.ds(base, rows_w)])


def _attn_body(top_ref, q_ref, k_ref, v_ref, o_ref, st0_ref, st1_ref):
    hp = pl.program_id(0)
    g = pl.program_id(1)
    nqb = 64
    for hh, st_ref in ((0, st0_ref), (1, st1_ref)):
        kh = k_ref[:, hh * HD:(hh + 1) * HD]
        qh = q_ref[:, hh * HD:(hh + 1) * HD]
        st_ref[...] = _mm(kh, qh, (((1,), (1,)), ((), ())))
    for j in range(QBS):
        qb = g * QBS + j
        outs = []
        for hh, st_ref in ((0, st0_ref), (1, st1_ref)):
            row = (hp * 2 + hh) * nqb + qb
            ss, vs = [], []
            for m in range(NBLK):
                idx = top_ref[row, m]
                ss.append(st_ref[pl.ds(idx * BLK, BLK), j * BLK:(j + 1) * BLK])
                vs.append(v_ref[pl.ds(idx * BLK, BLK), :][:, hh * HD:(hh + 1) * HD])
            s = jnp.concatenate(ss, axis=0)
            vsel = jnp.concatenate(vs, axis=0)
            mx = jnp.max(s, axis=0, keepdims=True)
            e = jnp.exp(s - mx)
            p = e / jnp.sum(e, axis=0, keepdims=True)
            outs.append(_mm(p, vsel, (((0,), (0,)), ((), ()))))
        o_ref[j * BLK:(j + 1) * BLK, :] = jnp.concatenate(outs, axis=1)


def _head_body(ctx_ref, wo_ref, bo_ref, wp_ref, bp_ref, ws_ref, wc_ref, bc_ref,
               out_ref):
    ctx = ctx_ref[...]
    ao = _mm(ctx, wo_ref[...]) + bo_ref[...]
    hp = jnp.tanh(_mm(ao, wp_ref[...]) + bp_ref[...])
    sc = jax.lax.dot_general(hp, ws_ref[...], (((1,), (1,)), ((), ())),
                             preferred_element_type=jnp.float32)
    mx = jnp.max(sc, axis=0, keepdims=True)
    e = jnp.exp(sc - mx)
    p = e / jnp.sum(e, axis=0, keepdims=True)
    pooled = jax.lax.dot_general(p, ao, (((0,), (0,)), ((), ())),
                                 preferred_element_type=jnp.float32)
    out_ref[...] = jax.lax.dot_general(pooled, wc_ref[...],
                                       (((1,), (0,)), ((), ())),
                                       preferred_element_type=jnp.float32) + bc_ref[...]


def kernel(hidden_states, attention_mask, Wq, bq, Wk, bk, Wv, bv, Wo, bo,
           Wp, bp, ws, Wc, bc):
    del attention_mask
    bsz, T, DM = hidden_states.shape
    hs = hidden_states.reshape(T, DM)
    nqb = T // BLK
    nlab = Wc.shape[1]
    bq2 = bq.reshape(1, DM)
    bk2 = bk.reshape(1, DM)
    bv2 = bv.reshape(1, DM)
    bo2 = bo.reshape(1, DM)
    bp2 = bp.reshape(1, DM)
    ws2 = ws.reshape(1, DM)
    bc2 = bc.reshape(1, nlab)

    ntile = T // SEQ_TILE
    nb_tile = SEQ_TILE // BLK
    q, k, v, qbm, kbm = pl.pallas_call(
        _qkv_body,
        grid=(ntile,),
        in_specs=[
            pl.BlockSpec((SEQ_TILE, DM), lambda t: (t, 0)),
            pl.BlockSpec((DM, DM), lambda t: (0, 0)),
            pl.BlockSpec((DM, DM), lambda t: (0, 0)),
            pl.BlockSpec((DM, DM), lambda t: (0, 0)),
            pl.BlockSpec((1, DM), lambda t: (0, 0)),
            pl.BlockSpec((1, DM), lambda t: (0, 0)),
            pl.BlockSpec((1, DM), lambda t: (0, 0)),
        ],
        out_specs=[
            pl.BlockSpec((SEQ_TILE, DM), lambda t: (t, 0)),
            pl.BlockSpec((SEQ_TILE, DM), lambda t: (t, 0)),
            pl.BlockSpec((SEQ_TILE, DM), lambda t: (t, 0)),
            pl.BlockSpec((nb_tile, DM), lambda t: (t, 0)),
            pl.BlockSpec((nb_tile, DM), lambda t: (t, 0)),
        ],
        out_shape=[
            jax.ShapeDtypeStruct((T, DM), jnp.float32),
            jax.ShapeDtypeStruct((T, DM), jnp.float32),
            jax.ShapeDtypeStruct((T, DM), jnp.float32),
            jax.ShapeDtypeStruct((nqb, DM), jnp.float32),
            jax.ShapeDtypeStruct((nqb, DM), jnp.float32),
        ],
    )(hs, Wq, Wk, Wv, bq2, bk2, bv2)

    scores = pl.pallas_call(
        _scores_body,
        out_shape=jax.ShapeDtypeStruct((H * nqb, nqb), jnp.float32),
    )(qbm, kbm)

    rows_w = (H * nqb) // _NW
    top4 = pl.kernel(
        _topk_sc_body,
        out_type=jax.ShapeDtypeStruct((H * nqb, 16), jnp.int32),
        mesh=plsc.VectorSubcoreMesh(core_axis_name="c", subcore_axis_name="s",
                                    num_cores=_NC, num_subcores=_NS),
        scratch_types=[
            pltpu.VMEM((rows_w, nqb), jnp.float32),
            pltpu.VMEM((rows_w, 16), jnp.int32),
            pltpu.VMEM((32,), jnp.float32),
            pltpu.VMEM((32,), jnp.int32),
        ],
    )(scores)

    grid_spec = pltpu.PrefetchScalarGridSpec(
        num_scalar_prefetch=1,
        grid=(H // 2, nqb // QBS),
        in_specs=[
            pl.BlockSpec((QBS * BLK, 2 * HD), lambda hp, g, top: (g, hp)),
            pl.BlockSpec((T, 2 * HD), lambda hp, g, top: (0, hp)),
            pl.BlockSpec((T, 2 * HD), lambda hp, g, top: (0, hp)),
        ],
        out_specs=pl.BlockSpec((QBS * BLK, 2 * HD), lambda hp, g, top: (g, hp)),
        scratch_shapes=[
            pltpu.VMEM((T, QBS * BLK), jnp.float32),
            pltpu.VMEM((T, QBS * BLK), jnp.float32),
        ],
    )
    ctx = pl.pallas_call(
        _attn_body,
        grid_spec=grid_spec,
        out_shape=jax.ShapeDtypeStruct((T, DM), jnp.float32),
    )(top4, q, k, v)

    logits = pl.pallas_call(
        _head_body,
        out_shape=jax.ShapeDtypeStruct((1, nlab), jnp.float32),
    )(ctx, Wo, bo2, Wp, bp2, ws2, Wc, bc2)
    return logits


# speedup vs baseline: 2.9204x; 1.0505x over previous
import jax
import jax.numpy as jnp
from jax.experimental import pallas as pl
from jax.experimental.pallas import tpu as pltpu
from jax.experimental.pallas import tpu_sc as plsc

H = 12
HD = 64
BLK = 32
NBLK = 4
QBS = 32
SEQ_TILE = 256

_NC = 2
_NS = 16
_NW = _NC * _NS


def _mm(a, b, dims=None):
    if dims is None:
        dims = (((a.ndim - 1,), (0,)), ((), ()))
    return jax.lax.dot_general(a.astype(jnp.bfloat16), b.astype(jnp.bfloat16),
                               dims, preferred_element_type=jnp.float32)


def _proj_body(hs_ref, wq_ref, wk_ref, wv_ref, bq_ref, bk_ref, bv_ref,
               q_ref, k_ref, v_ref, s_ref, qbs_ref, kbs_ref):
    t = pl.program_id(0)
    nt = pl.num_programs(0)
    hs = hs_ref[...]
    scale = HD ** -0.5
    q = (_mm(hs, wq_ref[...]) + bq_ref[...]) * scale
    k = _mm(hs, wk_ref[...]) + bk_ref[...]
    v = _mm(hs, wv_ref[...]) + bv_ref[...]
    q_ref[...] = q.astype(jnp.bfloat16)
    k_ref[...] = k.astype(jnp.bfloat16)
    v_ref[...] = v.astype(jnp.bfloat16)
    ts, dm = hs.shape
    nb = ts // BLK
    qbs_ref[pl.ds(t * nb, nb), :] = jnp.mean(q.reshape(nb, BLK, dm), axis=1)
    kbs_ref[pl.ds(t * nb, nb), :] = jnp.mean(k.reshape(nb, BLK, dm), axis=1)

    @pl.when(t == nt - 1)
    def _():
        nqb = qbs_ref.shape[0]
        for h in range(H):
            qh = qbs_ref[:, h * HD:(h + 1) * HD]
            kh = kbs_ref[:, h * HD:(h + 1) * HD]
            s_ref[h * nqb:(h + 1) * nqb, :] = _mm(qh, kh, (((1,), (1,)), ((), ())))


def _topk_sc_body(scores_hbm, top_hbm, sc_v, top_v, rotf, roti):
    rows_w = sc_v.shape[0]
    nb = sc_v.shape[1]
    nchunk = nb // 16
    wid = jax.lax.axis_index("s") * _NC + jax.lax.axis_index("c")
    base = wid * rows_w
    pltpu.sync_copy(scores_hbm.at[pl.ds(base, rows_w)], sc_v)
    lane = jax.lax.iota(jnp.int32, 16)

    def _allmax(v):
        for sh in (8, 4, 2, 1):
            rotf[0:16] = v
            rotf[16:32] = v
            v = jnp.maximum(v, rotf[sh:sh + 16])
        return v

    def _allmin_i(v):
        for sh in (8, 4, 2, 1):
            roti[0:16] = v
            roti[16:32] = v
            v = jnp.minimum(v, roti[sh:sh + 16])
        return v

    def row_fn(r, carry):
        vals = [sc_v[r, 16 * c:16 * c + 16] for c in range(nchunk)]
        gidx = [lane + 16 * c for c in range(nchunk)]
        out = jnp.zeros((16,), jnp.int32)
        for m in range(NBLK):
            mx = vals[0]
            for c in range(1, nchunk):
                mx = jnp.maximum(mx, vals[c])
            mxv = _allmax(mx)
            cmin = jnp.where(vals[0] == mxv, gidx[0], nb)
            for c in range(1, nchunk):
                cmin = jnp.minimum(cmin, jnp.where(vals[c] == mxv, gidx[c], nb))
            civ = _allmin_i(cmin)
            out = jnp.where(lane == m, civ, out)
            for c in range(nchunk):
                vals[c] = jnp.where(gidx[c] == civ, -1e30, vals[c])
        top_v[r, :] = out
        return carry

    jax.lax.fori_loop(0, rows_w, row_fn, 0)
    pltpu.sync_copy(top_v, top_hbm.at[pl.ds(base, rows_w)])


def _attn_head_body(top_ref, q_ref, k_ref, v_ref, wo_ref, bo_ref, wp_ref,
                    bp_ref, ws_ref, wc_ref, bc_ref, out_ref,
                    st0_ref, st1_ref, ctx_ref):
    hp = pl.program_id(0)
    g = pl.program_id(1)
    ng = pl.num_programs(1)
    nqb = 64
    for hh, st_ref in ((0, st0_ref), (1, st1_ref)):
        kh = k_ref[:, hh * HD:(hh + 1) * HD]
        qh = q_ref[:, hh * HD:(hh + 1) * HD]
        st_ref[...] = _mm(kh, qh, (((1,), (1,)), ((), ())))
    for j in range(QBS):
        qb = g * QBS + j
        outs = []
        for hh, st_ref in ((0, st0_ref), (1, st1_ref)):
            row = (hp * 2 + hh) * nqb + qb
            ss, vs = [], []
            for m in range(NBLK):
                idx = top_ref[row, m]
                ss.append(st_ref[pl.ds(idx * BLK, BLK), j * BLK:(j + 1) * BLK])
                vs.append(v_ref[pl.ds(idx * BLK, BLK), :][:, hh * HD:(hh + 1) * HD])
            s = jnp.concatenate(ss, axis=0)
            vsel = jnp.concatenate(vs, axis=0)
            mx = jnp.max(s, axis=0, keepdims=True)
            e = jnp.exp(s - mx)
            p = e / jnp.sum(e, axis=0, keepdims=True)
            outs.append(_mm(p, vsel, (((0,), (0,)), ((), ()))))
        ctx_ref[hp, pl.ds(qb * BLK, BLK), :] = jnp.concatenate(outs, axis=1)

    @pl.when((hp == H // 2 - 1) & (g == ng - 1))
    def _():
        ctx = jnp.concatenate([ctx_ref[i] for i in range(H // 2)], axis=1)
        ao = _mm(ctx, wo_ref[...]) + bo_ref[...]
        hpool = jnp.tanh(_mm(ao, wp_ref[...]) + bp_ref[...])
        sc = jax.lax.dot_general(hpool, ws_ref[...], (((1,), (1,)), ((), ())),
                                 preferred_element_type=jnp.float32)
        mxs = jnp.max(sc, axis=0, keepdims=True)
        e = jnp.exp(sc - mxs)
        p = e / jnp.sum(e, axis=0, keepdims=True)
        pooled = jax.lax.dot_general(p, ao, (((0,), (0,)), ((), ())),
                                     preferred_element_type=jnp.float32)
        out_ref[...] = jax.lax.dot_general(pooled, wc_ref[...],
                                           (((1,), (0,)), ((), ())),
                                           preferred_element_type=jnp.float32) + bc_ref[...]


def kernel(hidden_states, attention_mask, Wq, bq, Wk, bk, Wv, bv, Wo, bo,
           Wp, bp, ws, Wc, bc):
    del attention_mask
    bsz, T, DM = hidden_states.shape
    hs = hidden_states.reshape(T, DM)
    nqb = T // BLK
    nlab = Wc.shape[1]
    bq2 = bq.reshape(1, DM)
    bk2 = bk.reshape(1, DM)
    bv2 = bv.reshape(1, DM)
    bo2 = bo.reshape(1, DM)
    bp2 = bp.reshape(1, DM)
    ws2 = ws.reshape(1, DM)
    bc2 = bc.reshape(1, nlab)

    ntile = T // SEQ_TILE
    q, k, v, scores = pl.pallas_call(
        _proj_body,
        grid=(ntile,),
        in_specs=[
            pl.BlockSpec((SEQ_TILE, DM), lambda t: (t, 0)),
            pl.BlockSpec((DM, DM), lambda t: (0, 0)),
            pl.BlockSpec((DM, DM), lambda t: (0, 0)),
            pl.BlockSpec((DM, DM), lambda t: (0, 0)),
            pl.BlockSpec((1, DM), lambda t: (0, 0)),
            pl.BlockSpec((1, DM), lambda t: (0, 0)),
            pl.BlockSpec((1, DM), lambda t: (0, 0)),
        ],
        out_specs=[
            pl.BlockSpec((SEQ_TILE, DM), lambda t: (t, 0)),
            pl.BlockSpec((SEQ_TILE, DM), lambda t: (t, 0)),
            pl.BlockSpec((SEQ_TILE, DM), lambda t: (t, 0)),
            pl.BlockSpec((H * nqb, nqb), lambda t: (0, 0)),
        ],
        out_shape=[
            jax.ShapeDtypeStruct((T, DM), jnp.bfloat16),
            jax.ShapeDtypeStruct((T, DM), jnp.bfloat16),
            jax.ShapeDtypeStruct((T, DM), jnp.bfloat16),
            jax.ShapeDtypeStruct((H * nqb, nqb), jnp.float32),
        ],
        scratch_shapes=[
            pltpu.VMEM((nqb, DM), jnp.float32),
            pltpu.VMEM((nqb, DM), jnp.float32),
        ],
    )(hs, Wq, Wk, Wv, bq2, bk2, bv2)

    rows_w = (H * nqb) // _NW
    top4 = pl.kernel(
        _topk_sc_body,
        out_type=jax.ShapeDtypeStruct((H * nqb, 16), jnp.int32),
        mesh=plsc.VectorSubcoreMesh(core_axis_name="c", subcore_axis_name="s",
                                    num_cores=_NC, num_subcores=_NS),
        scratch_types=[
            pltpu.VMEM((rows_w, nqb), jnp.float32),
            pltpu.VMEM((rows_w, 16), jnp.int32),
            pltpu.VMEM((32,), jnp.float32),
            pltpu.VMEM((32,), jnp.int32),
        ],
    )(scores)

    grid_spec = pltpu.PrefetchScalarGridSpec(
        num_scalar_prefetch=1,
        grid=(H // 2, nqb // QBS),
        in_specs=[
            pl.BlockSpec((QBS * BLK, 2 * HD), lambda hp, g, top: (g, hp)),
            pl.BlockSpec((T, 2 * HD), lambda hp, g, top: (0, hp)),
            pl.BlockSpec((T, 2 * HD), lambda hp, g, top: (0, hp)),
            pl.BlockSpec((DM, DM), lambda hp, g, top: (0, 0)),
            pl.BlockSpec((1, DM), lambda hp, g, top: (0, 0)),
            pl.BlockSpec((DM, DM), lambda hp, g, top: (0, 0)),
            pl.BlockSpec((1, DM), lambda hp, g, top: (0, 0)),
            pl.BlockSpec((1, DM), lambda hp, g, top: (0, 0)),
            pl.BlockSpec((DM, nlab), lambda hp, g, top: (0, 0)),
            pl.BlockSpec((1, nlab), lambda hp, g, top: (0, 0)),
        ],
        out_specs=pl.BlockSpec((1, nlab), lambda hp, g, top: (0, 0)),
        scratch_shapes=[
            pltpu.VMEM((T, QBS * BLK), jnp.float32),
            pltpu.VMEM((T, QBS * BLK), jnp.float32),
            pltpu.VMEM((H // 2, T, 2 * HD), jnp.float32),
        ],
    )
    logits = pl.pallas_call(
        _attn_head_body,
        grid_spec=grid_spec,
        out_shape=jax.ShapeDtypeStruct((1, nlab), jnp.float32),
    )(top4, q, k, v, Wo, bo2, Wp, bp2, ws2, Wc, bc2)
    return logits


# speedup vs baseline: 3.0075x; 1.0298x over previous
import jax
import jax.numpy as jnp
from jax.experimental import pallas as pl
from jax.experimental.pallas import tpu as pltpu
from jax.experimental.pallas import tpu_sc as plsc

H = 12
HD = 64
BLK = 32
NBLK = 4
QBS = 64
SEQ_TILE = 512

_NC = 2
_NS = 16
_NW = _NC * _NS


def _mm(a, b, dims=None):
    if dims is None:
        dims = (((a.ndim - 1,), (0,)), ((), ()))
    return jax.lax.dot_general(a.astype(jnp.bfloat16), b.astype(jnp.bfloat16),
                               dims, preferred_element_type=jnp.float32)


def _proj_body(hs_ref, wq_ref, wk_ref, wv_ref, bq_ref, bk_ref, bv_ref,
               q_ref, k_ref, v_ref, s_ref, qbs_ref, kbs_ref):
    t = pl.program_id(0)
    nt = pl.num_programs(0)
    hs = hs_ref[...]
    scale = HD ** -0.5
    q = (_mm(hs, wq_ref[...]) + bq_ref[...]) * scale
    k = _mm(hs, wk_ref[...]) + bk_ref[...]
    v = _mm(hs, wv_ref[...]) + bv_ref[...]
    q_ref[...] = q.astype(jnp.bfloat16)
    k_ref[...] = k.astype(jnp.bfloat16)
    v_ref[...] = v.astype(jnp.bfloat16)
    ts, dm = hs.shape
    nb = ts // BLK
    qbs_ref[pl.ds(t * nb, nb), :] = jnp.mean(q.reshape(nb, BLK, dm), axis=1)
    kbs_ref[pl.ds(t * nb, nb), :] = jnp.mean(k.reshape(nb, BLK, dm), axis=1)

    @pl.when(t == nt - 1)
    def _():
        nqb = qbs_ref.shape[0]
        for h in range(H):
            qh = qbs_ref[:, h * HD:(h + 1) * HD]
            kh = kbs_ref[:, h * HD:(h + 1) * HD]
            s_ref[h * nqb:(h + 1) * nqb, :] = _mm(qh, kh, (((1,), (1,)), ((), ())))


def _topk_sc_body(scores_hbm, top_hbm, sc_v, top_v, rotf, roti):
    rows_w = sc_v.shape[0]
    nb = sc_v.shape[1]
    nchunk = nb // 16
    wid = jax.lax.axis_index("s") * _NC + jax.lax.axis_index("c")
    base = wid * rows_w
    pltpu.sync_copy(scores_hbm.at[pl.ds(base, rows_w)], sc_v)
    lane = jax.lax.iota(jnp.int32, 16)

    def _allmax(v):
        for sh in (8, 4, 2, 1):
            rotf[0:16] = v
            rotf[16:32] = v
            v = jnp.maximum(v, rotf[sh:sh + 16])
        return v

    def _allmin_i(v):
        for sh in (8, 4, 2, 1):
            roti[0:16] = v
            roti[16:32] = v
            v = jnp.minimum(v, roti[sh:sh + 16])
        return v

    def row_fn(r, carry):
        vals = [sc_v[r, 16 * c:16 * c + 16] for c in range(nchunk)]
        gidx = [lane + 16 * c for c in range(nchunk)]
        out = jnp.zeros((16,), jnp.int32)
        for m in range(NBLK):
            mx = vals[0]
            for c in range(1, nchunk):
                mx = jnp.maximum(mx, vals[c])
            mxv = _allmax(mx)
            cmin = jnp.where(vals[0] == mxv, gidx[0], nb)
            for c in range(1, nchunk):
                cmin = jnp.minimum(cmin, jnp.where(vals[c] == mxv, gidx[c], nb))
            civ = _allmin_i(cmin)
            out = jnp.where(lane == m, civ, out)
            for c in range(nchunk):
                vals[c] = jnp.where(gidx[c] == civ, -1e30, vals[c])
        top_v[r, :] = out
        return carry

    jax.lax.fori_loop(0, rows_w, row_fn, 0)
    pltpu.sync_copy(top_v, top_hbm.at[pl.ds(base, rows_w)])


def _attn_head_body(top_ref, q_ref, k_ref, v_ref, wo_ref, bo_ref, wp_ref,
                    bp_ref, ws_ref, wc_ref, bc_ref, out_ref,
                    st0_ref, st1_ref, ctx_ref):
    hp = pl.program_id(0)
    g = pl.program_id(1)
    ng = pl.num_programs(1)
    nqb = 64
    for hh, st_ref in ((0, st0_ref), (1, st1_ref)):
        kh = k_ref[:, hh * HD:(hh + 1) * HD]
        qh = q_ref[:, hh * HD:(hh + 1) * HD]
        st_ref[...] = _mm(kh, qh, (((1,), (1,)), ((), ())))
    for j in range(QBS):
        qb = g * QBS + j
        outs = []
        for hh, st_ref in ((0, st0_ref), (1, st1_ref)):
            row = (hp * 2 + hh) * nqb + qb
            ss, vs = [], []
            for m in range(NBLK):
                idx = top_ref[row, m]
                ss.append(st_ref[pl.ds(idx * BLK, BLK), j * BLK:(j + 1) * BLK])
                vs.append(v_ref[pl.ds(idx * BLK, BLK), :][:, hh * HD:(hh + 1) * HD])
            s = jnp.concatenate(ss, axis=0)
            vsel = jnp.concatenate(vs, axis=0)
            mx = jnp.max(s, axis=0, keepdims=True)
            e = jnp.exp(s - mx)
            p = e / jnp.sum(e, axis=0, keepdims=True)
            outs.append(_mm(p, vsel, (((0,), (0,)), ((), ()))))
        ctx_ref[hp, pl.ds(qb * BLK, BLK), :] = jnp.concatenate(outs, axis=1)

    @pl.when((hp == H // 2 - 1) & (g == ng - 1))
    def _():
        ctx = jnp.concatenate([ctx_ref[i] for i in range(H // 2)], axis=1)
        ao = _mm(ctx, wo_ref[...]) + bo_ref[...]
        hpool = jnp.tanh(_mm(ao, wp_ref[...]) + bp_ref[...])
        sc = jax.lax.dot_general(hpool, ws_ref[...], (((1,), (1,)), ((), ())),
                                 preferred_element_type=jnp.float32)
        mxs = jnp.max(sc, axis=0, keepdims=True)
        e = jnp.exp(sc - mxs)
        p = e / jnp.sum(e, axis=0, keepdims=True)
        pooled = jax.lax.dot_general(p, ao, (((0,), (0,)), ((), ())),
                                     preferred_element_type=jnp.float32)
        out_ref[...] = jax.lax.dot_general(pooled, wc_ref[...],
                                           (((1,), (0,)), ((), ())),
                                           preferred_element_type=jnp.float32) + bc_ref[...]


def kernel(hidden_states, attention_mask, Wq, bq, Wk, bk, Wv, bv, Wo, bo,
           Wp, bp, ws, Wc, bc):
    del attention_mask
    bsz, T, DM = hidden_states.shape
    hs = hidden_states.reshape(T, DM)
    nqb = T // BLK
    nlab = Wc.shape[1]
    bq2 = bq.reshape(1, DM)
    bk2 = bk.reshape(1, DM)
    bv2 = bv.reshape(1, DM)
    bo2 = bo.reshape(1, DM)
    bp2 = bp.reshape(1, DM)
    ws2 = ws.reshape(1, DM)
    bc2 = bc.reshape(1, nlab)

    ntile = T // SEQ_TILE
    q, k, v, scores = pl.pallas_call(
        _proj_body,
        grid=(ntile,),
        in_specs=[
            pl.BlockSpec((SEQ_TILE, DM), lambda t: (t, 0)),
            pl.BlockSpec((DM, DM), lambda t: (0, 0)),
            pl.BlockSpec((DM, DM), lambda t: (0, 0)),
            pl.BlockSpec((DM, DM), lambda t: (0, 0)),
            pl.BlockSpec((1, DM), lambda t: (0, 0)),
            pl.BlockSpec((1, DM), lambda t: (0, 0)),
            pl.BlockSpec((1, DM), lambda t: (0, 0)),
        ],
        out_specs=[
            pl.BlockSpec((SEQ_TILE, DM), lambda t: (t, 0)),
            pl.BlockSpec((SEQ_TILE, DM), lambda t: (t, 0)),
            pl.BlockSpec((SEQ_TILE, DM), lambda t: (t, 0)),
            pl.BlockSpec((H * nqb, nqb), lambda t: (0, 0)),
        ],
        out_shape=[
            jax.ShapeDtypeStruct((T, DM), jnp.bfloat16),
            jax.ShapeDtypeStruct((T, DM), jnp.bfloat16),
            jax.ShapeDtypeStruct((T, DM), jnp.bfloat16),
            jax.ShapeDtypeStruct((H * nqb, nqb), jnp.float32),
        ],
        scratch_shapes=[
            pltpu.VMEM((nqb, DM), jnp.float32),
            pltpu.VMEM((nqb, DM), jnp.float32),
        ],
    )(hs, Wq, Wk, Wv, bq2, bk2, bv2)

    rows_w = (H * nqb) // _NW
    top4 = pl.kernel(
        _topk_sc_body,
        out_type=jax.ShapeDtypeStruct((H * nqb, 16), jnp.int32),
        mesh=plsc.VectorSubcoreMesh(core_axis_name="c", subcore_axis_name="s",
                                    num_cores=_NC, num_subcores=_NS),
        scratch_types=[
            pltpu.VMEM((rows_w, nqb), jnp.float32),
            pltpu.VMEM((rows_w, 16), jnp.int32),
            pltpu.VMEM((32,), jnp.float32),
            pltpu.VMEM((32,), jnp.int32),
        ],
    )(scores)

    grid_spec = pltpu.PrefetchScalarGridSpec(
        num_scalar_prefetch=1,
        grid=(H // 2, nqb // QBS),
        in_specs=[
            pl.BlockSpec((QBS * BLK, 2 * HD), lambda hp, g, top: (g, hp)),
            pl.BlockSpec((T, 2 * HD), lambda hp, g, top: (0, hp)),
            pl.BlockSpec((T, 2 * HD), lambda hp, g, top: (0, hp)),
            pl.BlockSpec((DM, DM), lambda hp, g, top: (0, 0)),
            pl.BlockSpec((1, DM), lambda hp, g, top: (0, 0)),
            pl.BlockSpec((DM, DM), lambda hp, g, top: (0, 0)),
            pl.BlockSpec((1, DM), lambda hp, g, top: (0, 0)),
            pl.BlockSpec((1, DM), lambda hp, g, top: (0, 0)),
            pl.BlockSpec((DM, nlab), lambda hp, g, top: (0, 0)),
            pl.BlockSpec((1, nlab), lambda hp, g, top: (0, 0)),
        ],
        out_specs=pl.BlockSpec((1, nlab), lambda hp, g, top: (0, 0)),
        scratch_shapes=[
            pltpu.VMEM((T, QBS * BLK), jnp.float32),
            pltpu.VMEM((T, QBS * BLK), jnp.float32),
            pltpu.VMEM((H // 2, T, 2 * HD), jnp.float32),
        ],
    )
    logits = pl.pallas_call(
        _attn_head_body,
        grid_spec=grid_spec,
        out_shape=jax.ShapeDtypeStruct((1, nlab), jnp.float32),
    )(top4, q, k, v, Wo, bo2, Wp, bp2, ws2, Wc, bc2)
    return logits
